# Initial kernel scaffold; baseline (speedup 1.0000x reference)
import jax
import jax.numpy as jnp
from jax.experimental import pallas as pl


def kernel(inputs, position_embeddings):
    raise NotImplementedError("write your pallas kernel here")


# speedup vs baseline: 1.0044x; 1.0044x over previous
import jax
import jax.numpy as jnp
from jax.experimental import pallas as pl


def _bcast_body(tab_ref, out_ref):
    out_ref[...] = tab_ref[...][None, :, :]


def kernel(inputs, position_embeddings):
    batch, seq_len, d_model = inputs.shape
    block_s = 512
    grid = (seq_len // block_s, batch)
    out = pl.pallas_call(
        _bcast_body,
        grid=grid,
        in_specs=[
            pl.BlockSpec((block_s, d_model), lambda i, b: (i, 0)),
        ],
        out_specs=pl.BlockSpec((1, block_s, d_model), lambda i, b: (b, i, 0)),
        out_shape=jax.ShapeDtypeStruct((batch, seq_len, d_model), position_embeddings.dtype),
    )(position_embeddings)
    return out
